# baseline (speedup 1.0000x reference)
import functools

import jax
import jax.numpy as jnp
from jax import lax
from jax.experimental import pallas as pl
from jax.experimental.pallas import tpu as pltpu
from jax.experimental.pallas import tpu_sc as plsc

NUM_CORES = 2
NUM_SUBCORES = 16
LANES = 16
NUM_WORKERS = NUM_CORES * NUM_SUBCORES

BATCH = 16384
EMB_DIM = 64
B_PER_W = BATCH // NUM_WORKERS
IDX_CHUNK = 128
N_CHUNKS = B_PER_W // IDX_CHUNK
GROUPS = B_PER_W // LANES


def _bpr_body(bu_hbm, bpi_hbm, bni_hbm, uemb_hbm, iemb_hbm,
              pos_hbm, neg_hbm,
              idx_u, idx_i, idx_j, rows_u, rows_i, rows_j,
              pos_v, neg_v, sem):
    wid = lax.axis_index("s") * NUM_CORES + lax.axis_index("c")
    base = wid * B_PER_W

    for k in range(N_CHUNKS):
        off = base + k * IDX_CHUNK
        pltpu.sync_copy(bu_hbm.at[pl.ds(off, IDX_CHUNK)], idx_u.at[k])
        pltpu.sync_copy(bpi_hbm.at[pl.ds(off, IDX_CHUNK)], idx_i.at[k])
        pltpu.sync_copy(bni_hbm.at[pl.ds(off, IDX_CHUNK)], idx_j.at[k])

    copies = []
    for k in range(N_CHUNKS):
        sl = pl.ds(k * IDX_CHUNK, IDX_CHUNK)
        copies.append(pltpu.async_copy(uemb_hbm.at[idx_u.at[k]],
                                       rows_u.at[sl], sem))
        copies.append(pltpu.async_copy(iemb_hbm.at[idx_i.at[k]],
                                       rows_i.at[sl], sem))
        copies.append(pltpu.async_copy(iemb_hbm.at[idx_j.at[k]],
                                       rows_j.at[sl], sem))
    for cp in copies:
        cp.wait()

    def group_body(g, carry):
        row = g * LANES + lax.iota(jnp.int32, LANES)
        accp = jnp.zeros((LANES,), jnp.float32)
        accn = jnp.zeros((LANES,), jnp.float32)
        col = jnp.zeros((LANES,), jnp.int32)
        ones = jnp.ones((LANES,), jnp.int32)
        for _ in range(EMB_DIM):
            u = plsc.load_gather(rows_u, [row, col])
            iv = plsc.load_gather(rows_i, [row, col])
            jv = plsc.load_gather(rows_j, [row, col])
            accp = accp + u * iv
            accn = accn + u * jv
            col = col + ones
        pos_v[pl.ds(g * LANES, LANES)] = accp
        neg_v[pl.ds(g * LANES, LANES)] = accn
        return carry

    lax.fori_loop(0, GROUPS, group_body, 0, unroll=False)

    pltpu.sync_copy(pos_v, pos_hbm.at[pl.ds(base, B_PER_W)])
    pltpu.sync_copy(neg_v, neg_hbm.at[pl.ds(base, B_PER_W)])


@jax.jit
def _bpr_scores(batch_user, batch_pos_item, batch_neg_item,
                user_emb, item_emb):
    mesh = plsc.VectorSubcoreMesh(core_axis_name="c", subcore_axis_name="s",
                                  num_cores=NUM_CORES,
                                  num_subcores=NUM_SUBCORES)
    run = pl.kernel(
        _bpr_body,
        out_type=[jax.ShapeDtypeStruct((BATCH,), jnp.float32),
                  jax.ShapeDtypeStruct((BATCH,), jnp.float32)],
        mesh=mesh,
        compiler_params=pltpu.CompilerParams(needs_layout_passes=False,
                                             use_tc_tiling_on_sc=False),
        scratch_types=[
            pltpu.VMEM((N_CHUNKS, IDX_CHUNK), jnp.int32),
            pltpu.VMEM((N_CHUNKS, IDX_CHUNK), jnp.int32),
            pltpu.VMEM((N_CHUNKS, IDX_CHUNK), jnp.int32),
            pltpu.VMEM((B_PER_W, EMB_DIM), jnp.float32),
            pltpu.VMEM((B_PER_W, EMB_DIM), jnp.float32),
            pltpu.VMEM((B_PER_W, EMB_DIM), jnp.float32),
            pltpu.VMEM((B_PER_W,), jnp.float32),
            pltpu.VMEM((B_PER_W,), jnp.float32),
            pltpu.SemaphoreType.DMA,
        ],
    )
    return run(batch_user, batch_pos_item, batch_neg_item,
               user_emb, item_emb)


def kernel(batch_user, batch_pos_item, batch_neg_item, user_emb, item_emb):
    pos, neg = _bpr_scores(batch_user.astype(jnp.int32),
                           batch_pos_item.astype(jnp.int32),
                           batch_neg_item.astype(jnp.int32),
                           user_emb, item_emb)
    return (pos[:, None], neg[:, None])
